# baseline (speedup 1.0000x reference)
import jax
import jax.numpy as jnp
import numpy as np
from jax.experimental import pallas as pl
from jax.experimental.pallas import tpu as pltpu

_D = 256
_K = 16
_NF = 5


def _index_points(points, idx):
    return jax.vmap(lambda p, i: p[i])(points, idx)


def _bn_kernel(res_ref, bnw_ref, bnb_ref, out_ref):
    res = res_ref[...]
    mean = jnp.mean(res, axis=(0, 1), keepdims=True)
    var = jnp.mean((res - mean) ** 2, axis=(0, 1), keepdims=True)
    out_ref[...] = (res - mean) / jnp.sqrt(var + 1e-5) * bnw_ref[...] + bnb_ref[...]


def kernel(xyz, feats, wq, wk, wv, g1w, g1b, g2w, g2b, pew, peb, bnw, bnb):
    dists = jnp.sum((xyz[:, :, None, :] - xyz[:, None, :, :]) ** 2, axis=-1)
    knn_idx = jnp.argsort(dists, axis=-1)[:, :, :_K]
    knn_xyz = _index_points(xyz, knn_idx)
    q_attn = feats @ wq
    k_attn = _index_points(feats @ wk, knn_idx)
    v_attn = _index_points(feats @ wv, knn_idx)
    pos_diff = xyz[:, :, None, :] - knn_xyz
    embeds = [pos_diff]
    for f in np.linspace(1.0, 2.0 ** _NF, _NF):
        embeds.append(jnp.sin(pos_diff * float(f)))
        embeds.append(jnp.cos(pos_diff * float(f)))
    pos_embed = jnp.concatenate(embeds, axis=-1)
    pos_encode = pos_embed @ pew + peb
    edges = q_attn[:, :, None, :] - k_attn + pos_encode
    attn = jax.nn.relu(edges @ g1w + g1b) @ g2w + g2b
    attn = jax.nn.softmax(attn, axis=-2)
    res = jnp.sum(attn * (v_attn + pos_encode), axis=2)
    res = res + feats

    out = pl.pallas_call(
        _bn_kernel,
        out_shape=jax.ShapeDtypeStruct(res.shape, res.dtype),
    )(res, bnw.reshape(1, 1, _D), bnb.reshape(1, 1, _D))
    return out


# speedup vs baseline: 8.0025x; 8.0025x over previous
import jax
import jax.numpy as jnp
import numpy as np
from jax import lax
from jax.experimental import pallas as pl
from jax.experimental.pallas import tpu as pltpu
from jax.experimental.pallas import tpu_sc as plsc

D = 256
K = 16
NF = 5
B = 4
N = 1024
BN = B * N
E = BN * K
TW = 2 * D
QW = 128
TBW = TW + QW
PEW = 40
NM = 33
_FREQS = [float(f) for f in np.linspace(1.0, 2.0 ** NF, NF)]

_S_NP = np.zeros((QW, PEW), np.float32)
for _m in range(NM):
    _S_NP[2 * _m, _m] = 1.0
    if _m < 3:
        _sgn = -1.0
    else:
        _fi, _r = divmod(_m - 3, 6)
        _sgn = -1.0 if _r < 3 else 1.0
    _S_NP[2 * _m + 1, _m] = _sgn

ROWS_A = 256
NBLK_A = N // ROWS_A


def _knn_body(xyzs_ref, xyzt_ref, gidx_ref):
    step = pl.program_id(0)
    b = step // NBLK_A
    xi = xyzs_ref[0]
    xjt = xyzt_ref[0]
    acc = jnp.zeros((ROWS_A, N), jnp.float32)
    for c in range(3):
        dif = xi[:, c:c + 1] - xjt[c:c + 1, :]
        acc = acc + dif * dif
    kb = lax.bitcast_convert_type(acc, jnp.int32)
    jidx = lax.broadcasted_iota(jnp.int32, (ROWS_A, N), 1)
    keys = (kb & jnp.int32(-1024)) | jidx
    cols = []
    base = b * N
    for _ in range(K):
        m = jnp.min(keys, axis=1, keepdims=True)
        cols.append((m & 1023) + base)
        keys = jnp.where(keys == m, jnp.int32(0x7FFFFFFF), keys)
    gidx_ref[...] = jnp.concatenate(cols, axis=1)


def _knn_call(xyzs, xyzt):
    return pl.pallas_call(
        _knn_body,
        grid=(B * NBLK_A,),
        in_specs=[
            pl.BlockSpec((1, ROWS_A, 3), lambda s: (s // NBLK_A, s % NBLK_A, 0)),
            pl.BlockSpec((1, 3, N), lambda s: (s // NBLK_A, 0, 0)),
        ],
        out_specs=pl.BlockSpec((ROWS_A, K), lambda s: (s, 0)),
        out_shape=jax.ShapeDtypeStruct((BN, K), jnp.int32),
    )(xyzs, xyzt)


PKR = BN * 3 // 128


def _sincos_body(x_ref, s_ref, c_ref):
    x = x_ref[...]
    for i, f in enumerate(_FREQS):
        xf = x * f
        s_ref[PKR * i:PKR * (i + 1), :] = jnp.sin(xf)
        c_ref[PKR * i:PKR * (i + 1), :] = jnp.cos(xf)


def _sincos_call(xpk):
    return pl.pallas_call(
        _sincos_body,
        out_shape=[
            jax.ShapeDtypeStruct((NF * PKR, 128), jnp.float32),
            jax.ShapeDtypeStruct((NF * PKR, 128), jnp.float32),
        ],
    )(xpk)


ROWS_B = 256
NSTEP_B = BN // ROWS_B


def _proj_body(feats_ref, q66_ref, wq_ref, wk_ref, wv_ref, g1w_ref, g1b_ref,
               peb_ref, pewp_ref, smat_ref, qgc_ref, t1_ref, w66_ref,
               wqg_s, wkg_s, c1_s):
    step = pl.program_id(0)

    @pl.when(step == 0)
    def _init():
        g1w = g1w_ref[...]
        wqg_s[...] = wq_ref[...] @ g1w
        wkg_s[...] = wk_ref[...] @ g1w
        c1_s[...] = peb_ref[...] @ g1w + g1b_ref[...]
        smat = smat_ref[...]
        w66_ref[:, :D] = smat @ (pewp_ref[...] @ g1w)
        w66_ref[:, D:] = smat @ pewp_ref[...]

    f = feats_ref[...]
    qgc_ref[...] = f @ wqg_s[...] + c1_s[...]
    t1_ref[:, :D] = f @ wkg_s[...]
    t1_ref[:, D:TW] = f @ wv_ref[...]
    t1_ref[:, TW:] = q66_ref[...]


def _proj_call(feats2d, q66, wq, wk, wv, g1w, g1b, peb, pewp, smat):
    return pl.pallas_call(
        _proj_body,
        grid=(NSTEP_B,),
        in_specs=[
            pl.BlockSpec((ROWS_B, D), lambda s: (s, 0)),
            pl.BlockSpec((ROWS_B, QW), lambda s: (s, 0)),
            pl.BlockSpec((D, D), lambda s: (0, 0)),
            pl.BlockSpec((D, D), lambda s: (0, 0)),
            pl.BlockSpec((D, D), lambda s: (0, 0)),
            pl.BlockSpec((D, D), lambda s: (0, 0)),
            pl.BlockSpec((1, D), lambda s: (0, 0)),
            pl.BlockSpec((1, D), lambda s: (0, 0)),
            pl.BlockSpec((PEW, D), lambda s: (0, 0)),
            pl.BlockSpec((QW, PEW), lambda s: (0, 0)),
        ],
        out_specs=[
            pl.BlockSpec((ROWS_B, D), lambda s: (s, 0)),
            pl.BlockSpec((ROWS_B, TBW), lambda s: (s, 0)),
            pl.BlockSpec((QW, TW), lambda s: (0, 0)),
        ],
        out_shape=[
            jax.ShapeDtypeStruct((BN, D), jnp.float32),
            jax.ShapeDtypeStruct((BN, TBW), jnp.float32),
            jax.ShapeDtypeStruct((QW, TW), jnp.float32),
        ],
        scratch_shapes=[
            pltpu.VMEM((D, D), jnp.float32),
            pltpu.VMEM((D, D), jnp.float32),
            pltpu.VMEM((1, D), jnp.float32),
        ],
    )(feats2d, q66, wq, wk, wv, g1w, g1b, peb, pewp, smat)


NWORK = 32
CHUNK = 128
NCHUNK = E // (NWORK * CHUNK)


def _gather_body(t1_hbm, gidx_hbm, g1_out, idx_v, r1_v, sem1):
    c = lax.axis_index("c")
    s = lax.axis_index("s")
    wid = s * 2 + c

    def body(i, carry):
        base = (wid * NCHUNK + i) * CHUNK
        pltpu.sync_copy(gidx_hbm.at[pl.ds(base, CHUNK)], idx_v)
        pltpu.async_copy(t1_hbm.at[idx_v], r1_v, sem1).wait()
        pltpu.sync_copy(r1_v, g1_out.at[pl.ds(base, CHUNK)])
        return carry

    lax.fori_loop(0, NCHUNK, body, 0)


def _gather_call(t1, gidx):
    k = pl.kernel(
        _gather_body,
        out_type=jax.ShapeDtypeStruct((E, TBW), jnp.float32),
        mesh=plsc.VectorSubcoreMesh(core_axis_name="c", subcore_axis_name="s",
                                    num_cores=2, num_subcores=16),
        scratch_types=[
            pltpu.VMEM((CHUNK,), jnp.int32),
            pltpu.VMEM((CHUNK, TBW), jnp.float32),
            pltpu.SemaphoreType.DMA,
        ],
    )
    return k(t1, gidx)


P = 64
EP = P * K
NSTEP_C = BN // P


def _edge_body(qgc_ref, g1_ref, pfeat_ref, feats_ref, w66_ref,
               g2w_ref, peb_ref, res_ref, stats_ref, acc_s):
    step = pl.program_id(0)

    q = qgc_ref[...]
    qr = jnp.broadcast_to(q.reshape(P, 1, D), (P, K, D)).reshape(EP, D)
    pf = pfeat_ref[...]
    pfr = jnp.broadcast_to(pf.reshape(P, 1, QW), (P, K, QW)).reshape(EP, QW)
    g1 = g1_ref[...]
    prod = pfr * g1[:, TW:]

    pe2 = jax.lax.dot(prod, w66_ref[...],
                      preferred_element_type=jnp.float32)

    h = jnp.maximum(qr - g1[:, :D] + pe2[:, :D], 0.0)
    a = jax.lax.dot(h, g2w_ref[...], preferred_element_type=jnp.float32)

    a3 = a.reshape(P, K, D)
    m = jnp.max(a3, axis=1, keepdims=True)
    e = jnp.exp(a3 - m)
    ssum = jnp.sum(e, axis=1)
    v3 = (g1[:, D:TW] + pe2[:, D:]).reshape(P, K, D)
    num = jnp.sum(e * v3, axis=1)
    res = num / ssum + peb_ref[...] + feats_ref[...]
    res_ref[...] = res

    @pl.when(step == 0)
    def _init():
        acc_s[...] = jnp.zeros((8, D), jnp.float32)

    acc_s[0:1, :] += jnp.sum(res, axis=0, keepdims=True)
    acc_s[1:2, :] += jnp.sum(res * res, axis=0, keepdims=True)

    @pl.when(step == NSTEP_C - 1)
    def _fin():
        stats_ref[...] = acc_s[...]


def _edge_call(qgc, g1, pfeat, feats2d, w66, g2w, peb):
    return pl.pallas_call(
        _edge_body,
        grid=(NSTEP_C,),
        in_specs=[
            pl.BlockSpec((P, D), lambda s: (s, 0)),
            pl.BlockSpec((EP, TBW), lambda s: (s, 0)),
            pl.BlockSpec((P, QW), lambda s: (s, 0)),
            pl.BlockSpec((P, D), lambda s: (s, 0)),
            pl.BlockSpec((QW, TW), lambda s: (0, 0)),
            pl.BlockSpec((D, D), lambda s: (0, 0)),
            pl.BlockSpec((1, D), lambda s: (0, 0)),
        ],
        out_specs=[
            pl.BlockSpec((P, D), lambda s: (s, 0)),
            pl.BlockSpec((8, D), lambda s: (0, 0)),
        ],
        out_shape=[
            jax.ShapeDtypeStruct((BN, D), jnp.float32),
            jax.ShapeDtypeStruct((8, D), jnp.float32),
        ],
        scratch_shapes=[pltpu.VMEM((8, D), jnp.float32)],
    )(qgc, g1, pfeat, feats2d, w66, g2w, peb)


ROWS_D = 256
NSTEP_D = BN // ROWS_D


def _bn_body(res_ref, stats_ref, bnw_ref, bnb_ref, out_ref):
    inv_n = jnp.float32(1.0 / BN)
    mean = stats_ref[0:1, :] * inv_n
    ex2 = stats_ref[1:2, :] * inv_n
    var = ex2 - mean * mean
    scale = lax.rsqrt(var + 1e-5) * bnw_ref[...]
    out_ref[...] = (res_ref[...] - mean) * scale + bnb_ref[...]


def _bn_call(res, stats, bnw, bnb):
    return pl.pallas_call(
        _bn_body,
        grid=(NSTEP_D,),
        in_specs=[
            pl.BlockSpec((ROWS_D, D), lambda s: (s, 0)),
            pl.BlockSpec((8, D), lambda s: (0, 0)),
            pl.BlockSpec((1, D), lambda s: (0, 0)),
            pl.BlockSpec((1, D), lambda s: (0, 0)),
        ],
        out_specs=pl.BlockSpec((ROWS_D, D), lambda s: (s, 0)),
        out_shape=jax.ShapeDtypeStruct((BN, D), jnp.float32),
    )(res, stats, bnw, bnb)


def _assemble_pq(xyz3, s5, c5):
    ones = jnp.ones((BN,), jnp.float32)
    pcols, qcols = [], []
    for cdim in range(3):
        pcols += [xyz3[:, cdim], ones]
        qcols += [ones, xyz3[:, cdim]]
    for fi in range(NF):
        for cdim in range(3):
            pcols += [s5[fi, :, cdim], c5[fi, :, cdim]]
            qcols += [c5[fi, :, cdim], s5[fi, :, cdim]]
        for cdim in range(3):
            pcols += [c5[fi, :, cdim], s5[fi, :, cdim]]
            qcols += [c5[fi, :, cdim], s5[fi, :, cdim]]
    pmat = jnp.stack(pcols, axis=1)
    qmat = jnp.stack(qcols, axis=1)
    pad = jnp.zeros((BN, QW - 2 * NM), jnp.float32)
    return (jnp.concatenate([pmat, pad], axis=1),
            jnp.concatenate([qmat, pad], axis=1))


def kernel(xyz, feats, wq, wk, wv, g1w, g1b, g2w, g2b, pew, peb, bnw, bnb):
    del g2b
    xyzt = jnp.transpose(xyz, (0, 2, 1))
    gidx = _knn_call(xyz, xyzt).reshape(E)

    xyz3 = xyz.reshape(BN, 3)
    xpk = xyz3.reshape(PKR, 128)
    spk, cpk = _sincos_call(xpk)
    s5 = spk.reshape(NF, BN, 3)
    c5 = cpk.reshape(NF, BN, 3)
    pfeat, q66 = _assemble_pq(xyz3, s5, c5)

    feats2d = feats.reshape(BN, D)
    pewp = jnp.zeros((PEW, D), jnp.float32).at[:NM].set(pew)
    smat = jnp.asarray(_S_NP)
    qgc, t1, w66 = _proj_call(feats2d, q66, wq, wk, wv, g1w,
                              g1b.reshape(1, D), peb.reshape(1, D),
                              pewp, smat)
    g1 = _gather_call(t1, gidx)

    res, stats = _edge_call(qgc, g1, pfeat, feats2d, w66, g2w,
                            peb.reshape(1, D))
    out = _bn_call(res, stats, bnw.reshape(1, D), bnb.reshape(1, D))
    return out.reshape(B, N, D)


# speedup vs baseline: 12.2313x; 1.5284x over previous
import jax
import jax.numpy as jnp
import numpy as np
from jax import lax
from jax.experimental import pallas as pl
from jax.experimental.pallas import tpu as pltpu
from jax.experimental.pallas import tpu_sc as plsc

D = 256
K = 16
NF = 5
B = 4
N = 1024
BN = B * N
E = BN * K
TW = 2 * D
QW = 128
TBW = TW + QW
PEW = 40
NM = 33
_FREQS = [float(f) for f in np.linspace(1.0, 2.0 ** NF, NF)]

_S_NP = np.zeros((QW, PEW), np.float32)
for _c in range(3):
    _S_NP[_c, _c] = 1.0
    _S_NP[3 + _c, _c] = -1.0
for _fi in range(NF):
    _base = 6 + 12 * _fi
    for _c in range(3):
        _ms, _mc = 3 + 6 * _fi + _c, 3 + 6 * _fi + 3 + _c
        _S_NP[_base + _c, _ms] = 1.0
        _S_NP[_base + 3 + _c, _ms] = -1.0
        _S_NP[_base + 6 + _c, _mc] = 1.0
        _S_NP[_base + 9 + _c, _mc] = 1.0

ROWS_A = 256
NBLK_A = N // ROWS_A


def _knn_body(xyzs_ref, xyzt_ref, gidx_ref):
    step = pl.program_id(0)
    b = step // NBLK_A
    xi = xyzs_ref[0]
    xjt = xyzt_ref[0]
    acc = jnp.zeros((ROWS_A, N), jnp.float32)
    for c in range(3):
        dif = xi[:, c:c + 1] - xjt[c:c + 1, :]
        acc = acc + dif * dif
    kb = lax.bitcast_convert_type(acc, jnp.int32)
    jidx = lax.broadcasted_iota(jnp.int32, (ROWS_A, N), 1)
    keys = (kb & jnp.int32(-1024)) | jidx
    cols = []
    base = b * N
    for _ in range(K):
        m = jnp.min(keys, axis=1, keepdims=True)
        cols.append((m & 1023) + base)
        keys = jnp.where(keys == m, jnp.int32(0x7FFFFFFF), keys)
    gidx_ref[...] = jnp.concatenate(cols, axis=1)


def _knn_call(xyzs, xyzt):
    return pl.pallas_call(
        _knn_body,
        grid=(B * NBLK_A,),
        in_specs=[
            pl.BlockSpec((1, ROWS_A, 3), lambda s: (s // NBLK_A, s % NBLK_A, 0)),
            pl.BlockSpec((1, 3, N), lambda s: (s // NBLK_A, 0, 0)),
        ],
        out_specs=pl.BlockSpec((ROWS_A, K), lambda s: (s, 0)),
        out_shape=jax.ShapeDtypeStruct((BN, K), jnp.int32),
    )(xyzs, xyzt)


def _pq_body(xt_ref, p_ref, q_ref, tp_s, tq_s):
    x = xt_ref[...]
    one = jnp.ones((3, BN), jnp.float32)
    tp_s[0:3, :] = x
    tp_s[3:6, :] = one
    tq_s[0:3, :] = one
    tq_s[3:6, :] = x
    for fi, f in enumerate(_FREQS):
        s = jnp.sin(x * f)
        c = jnp.cos(x * f)
        base = 6 + 12 * fi
        tp_s[base:base + 3, :] = s
        tp_s[base + 3:base + 6, :] = c
        tp_s[base + 6:base + 9, :] = c
        tp_s[base + 9:base + 12, :] = s
        tq_s[base:base + 3, :] = c
        tq_s[base + 3:base + 6, :] = s
        tq_s[base + 6:base + 9, :] = c
        tq_s[base + 9:base + 12, :] = s
    zpad = jnp.zeros((QW - 2 * NM, BN), jnp.float32)
    tp_s[2 * NM:, :] = zpad
    tq_s[2 * NM:, :] = zpad
    p_ref[...] = tp_s[...].T
    q_ref[...] = tq_s[...].T


def _pq_call(xt3):
    return pl.pallas_call(
        _pq_body,
        out_shape=[
            jax.ShapeDtypeStruct((BN, QW), jnp.float32),
            jax.ShapeDtypeStruct((BN, QW), jnp.float32),
        ],
        scratch_shapes=[
            pltpu.VMEM((QW, BN), jnp.float32),
            pltpu.VMEM((QW, BN), jnp.float32),
        ],
    )(xt3)


ROWS_B = 256
NSTEP_B = BN // ROWS_B


def _proj_body(feats_ref, q66_ref, wq_ref, wk_ref, wv_ref, g1w_ref, g1b_ref,
               peb_ref, pewp_ref, smat_ref, qgc_ref, t1_ref, w66_ref,
               wqg_s, wkg_s, c1_s):
    step = pl.program_id(0)

    @pl.when(step == 0)
    def _init():
        g1w = g1w_ref[...]
        wqg_s[...] = wq_ref[...] @ g1w
        wkg_s[...] = wk_ref[...] @ g1w
        c1_s[...] = peb_ref[...] @ g1w + g1b_ref[...]
        smat = smat_ref[...]
        w66_ref[:, :D] = smat @ (pewp_ref[...] @ g1w)
        w66_ref[:, D:] = smat @ pewp_ref[...]

    f = feats_ref[...]
    qgc_ref[...] = f @ wqg_s[...] + c1_s[...]
    t1_ref[:, :D] = f @ wkg_s[...]
    t1_ref[:, D:TW] = f @ wv_ref[...]
    t1_ref[:, TW:] = q66_ref[...]


def _proj_call(feats2d, q66, wq, wk, wv, g1w, g1b, peb, pewp, smat):
    return pl.pallas_call(
        _proj_body,
        grid=(NSTEP_B,),
        in_specs=[
            pl.BlockSpec((ROWS_B, D), lambda s: (s, 0)),
            pl.BlockSpec((ROWS_B, QW), lambda s: (s, 0)),
            pl.BlockSpec((D, D), lambda s: (0, 0)),
            pl.BlockSpec((D, D), lambda s: (0, 0)),
            pl.BlockSpec((D, D), lambda s: (0, 0)),
            pl.BlockSpec((D, D), lambda s: (0, 0)),
            pl.BlockSpec((1, D), lambda s: (0, 0)),
            pl.BlockSpec((1, D), lambda s: (0, 0)),
            pl.BlockSpec((PEW, D), lambda s: (0, 0)),
            pl.BlockSpec((QW, PEW), lambda s: (0, 0)),
        ],
        out_specs=[
            pl.BlockSpec((ROWS_B, D), lambda s: (s, 0)),
            pl.BlockSpec((ROWS_B, TBW), lambda s: (s, 0)),
            pl.BlockSpec((QW, TW), lambda s: (0, 0)),
        ],
        out_shape=[
            jax.ShapeDtypeStruct((BN, D), jnp.float32),
            jax.ShapeDtypeStruct((BN, TBW), jnp.float32),
            jax.ShapeDtypeStruct((QW, TW), jnp.float32),
        ],
        scratch_shapes=[
            pltpu.VMEM((D, D), jnp.float32),
            pltpu.VMEM((D, D), jnp.float32),
            pltpu.VMEM((1, D), jnp.float32),
        ],
    )(feats2d, q66, wq, wk, wv, g1w, g1b, peb, pewp, smat)


NWORK = 32
CHUNK = 128
NCHUNK = E // (NWORK * CHUNK)


def _gather_body(t1_hbm, gidx_hbm, g1_out, idx_v, r1_v, sem1):
    c = lax.axis_index("c")
    s = lax.axis_index("s")
    wid = s * 2 + c

    def body(i, carry):
        base = (wid * NCHUNK + i) * CHUNK
        pltpu.sync_copy(gidx_hbm.at[pl.ds(base, CHUNK)], idx_v)
        pltpu.async_copy(t1_hbm.at[idx_v], r1_v, sem1).wait()
        pltpu.sync_copy(r1_v, g1_out.at[pl.ds(base, CHUNK)])
        return carry

    lax.fori_loop(0, NCHUNK, body, 0)


def _gather_call(t1, gidx):
    k = pl.kernel(
        _gather_body,
        out_type=jax.ShapeDtypeStruct((E, TBW), jnp.float32),
        mesh=plsc.VectorSubcoreMesh(core_axis_name="c", subcore_axis_name="s",
                                    num_cores=2, num_subcores=16),
        scratch_types=[
            pltpu.VMEM((CHUNK,), jnp.int32),
            pltpu.VMEM((CHUNK, TBW), jnp.float32),
            pltpu.SemaphoreType.DMA,
        ],
    )
    return k(t1, gidx)


P = 64
EP = P * K
NSTEP_C = BN // P


def _edge_body(qgc_ref, g1_ref, pfeat_ref, feats_ref, w66_ref,
               g2w_ref, peb_ref, res_ref, stats_ref, acc_s):
    step = pl.program_id(0)

    q = qgc_ref[...]
    qr = jnp.broadcast_to(q.reshape(P, 1, D), (P, K, D)).reshape(EP, D)
    pf = pfeat_ref[...]
    pfr = jnp.broadcast_to(pf.reshape(P, 1, QW), (P, K, QW)).reshape(EP, QW)
    g1 = g1_ref[...]
    prod = pfr * g1[:, TW:]

    pe2 = jax.lax.dot(prod, w66_ref[...],
                      preferred_element_type=jnp.float32)

    h = jnp.maximum(qr - g1[:, :D] + pe2[:, :D], 0.0)
    a = jax.lax.dot(h, g2w_ref[...], preferred_element_type=jnp.float32)

    a3 = a.reshape(P, K, D)
    m = jnp.max(a3, axis=1, keepdims=True)
    e = jnp.exp(a3 - m)
    ssum = jnp.sum(e, axis=1)
    v3 = (g1[:, D:TW] + pe2[:, D:]).reshape(P, K, D)
    num = jnp.sum(e * v3, axis=1)
    res = num / ssum + peb_ref[...] + feats_ref[...]
    res_ref[...] = res

    @pl.when(step == 0)
    def _init():
        acc_s[...] = jnp.zeros((8, D), jnp.float32)

    acc_s[0:1, :] += jnp.sum(res, axis=0, keepdims=True)
    acc_s[1:2, :] += jnp.sum(res * res, axis=0, keepdims=True)

    @pl.when(step == NSTEP_C - 1)
    def _fin():
        stats_ref[...] = acc_s[...]


def _edge_call(qgc, g1, pfeat, feats2d, w66, g2w, peb):
    return pl.pallas_call(
        _edge_body,
        grid=(NSTEP_C,),
        in_specs=[
            pl.BlockSpec((P, D), lambda s: (s, 0)),
            pl.BlockSpec((EP, TBW), lambda s: (s, 0)),
            pl.BlockSpec((P, QW), lambda s: (s, 0)),
            pl.BlockSpec((P, D), lambda s: (s, 0)),
            pl.BlockSpec((QW, TW), lambda s: (0, 0)),
            pl.BlockSpec((D, D), lambda s: (0, 0)),
            pl.BlockSpec((1, D), lambda s: (0, 0)),
        ],
        out_specs=[
            pl.BlockSpec((P, D), lambda s: (s, 0)),
            pl.BlockSpec((8, D), lambda s: (0, 0)),
        ],
        out_shape=[
            jax.ShapeDtypeStruct((BN, D), jnp.float32),
            jax.ShapeDtypeStruct((8, D), jnp.float32),
        ],
        scratch_shapes=[pltpu.VMEM((8, D), jnp.float32)],
    )(qgc, g1, pfeat, feats2d, w66, g2w, peb)


ROWS_D = 256
NSTEP_D = BN // ROWS_D


def _bn_body(res_ref, stats_ref, bnw_ref, bnb_ref, out_ref):
    inv_n = jnp.float32(1.0 / BN)
    mean = stats_ref[0:1, :] * inv_n
    ex2 = stats_ref[1:2, :] * inv_n
    var = ex2 - mean * mean
    scale = lax.rsqrt(var + 1e-5) * bnw_ref[...]
    out_ref[...] = (res_ref[...] - mean) * scale + bnb_ref[...]


def _bn_call(res, stats, bnw, bnb):
    return pl.pallas_call(
        _bn_body,
        grid=(NSTEP_D,),
        in_specs=[
            pl.BlockSpec((ROWS_D, D), lambda s: (s, 0)),
            pl.BlockSpec((8, D), lambda s: (0, 0)),
            pl.BlockSpec((1, D), lambda s: (0, 0)),
            pl.BlockSpec((1, D), lambda s: (0, 0)),
        ],
        out_specs=pl.BlockSpec((ROWS_D, D), lambda s: (s, 0)),
        out_shape=jax.ShapeDtypeStruct((BN, D), jnp.float32),
    )(res, stats, bnw, bnb)


def kernel(xyz, feats, wq, wk, wv, g1w, g1b, g2w, g2b, pew, peb, bnw, bnb):
    del g2b
    xyzt = jnp.transpose(xyz, (0, 2, 1))
    gidx = _knn_call(xyz, xyzt).reshape(E)

    xt3 = xyzt.transpose(1, 0, 2).reshape(3, BN)
    pfeat, q66 = _pq_call(xt3)

    feats2d = feats.reshape(BN, D)
    pewp = jnp.zeros((PEW, D), jnp.float32).at[:NM].set(pew)
    smat = jnp.asarray(_S_NP)
    qgc, t1, w66 = _proj_call(feats2d, q66, wq, wk, wv, g1w,
                              g1b.reshape(1, D), peb.reshape(1, D),
                              pewp, smat)
    g1 = _gather_call(t1, gidx)

    res, stats = _edge_call(qgc, g1, pfeat, feats2d, w66, g2w,
                            peb.reshape(1, D))
    out = _bn_call(res, stats, bnw.reshape(1, D), bnb.reshape(1, D))
    return out.reshape(B, N, D)


# speedup vs baseline: 14.9564x; 1.2228x over previous
import jax
import jax.numpy as jnp
import numpy as np
from jax import lax
from jax.experimental import pallas as pl
from jax.experimental.pallas import tpu as pltpu
from jax.experimental.pallas import tpu_sc as plsc

D = 256
K = 16
NF = 5
B = 4
N = 1024
BN = B * N
E = BN * K
TW = 2 * D
QW = 128
TBW = 384
PEW = 40
NM = 33
_FREQS = [float(f) for f in np.linspace(1.0, 2.0 ** NF, NF)]

_S_NP = np.zeros((QW, PEW), np.float32)
for _c in range(3):
    _S_NP[_c, _c] = 1.0
    _S_NP[3 + _c, _c] = -1.0
for _fi in range(NF):
    _base = 6 + 12 * _fi
    for _c in range(3):
        _ms, _mc = 3 + 6 * _fi + _c, 3 + 6 * _fi + 3 + _c
        _S_NP[_base + _c, _ms] = 1.0
        _S_NP[_base + 3 + _c, _ms] = -1.0
        _S_NP[_base + 6 + _c, _mc] = 1.0
        _S_NP[_base + 9 + _c, _mc] = 1.0

ROWS_A = 512
NBLK_A = N // ROWS_A


def _knn_body(xyzs_ref, xyzt_ref, gidx_ref):
    step = pl.program_id(0)
    b = step // NBLK_A
    xi = xyzs_ref[0]
    xjt = xyzt_ref[0]
    acc = jnp.zeros((ROWS_A, N), jnp.float32)
    for c in range(3):
        dif = xi[:, c:c + 1] - xjt[c:c + 1, :]
        acc = acc + dif * dif
    kb = lax.bitcast_convert_type(acc, jnp.int32)
    jidx = lax.broadcasted_iota(jnp.int32, (ROWS_A, N), 1)
    keys = (kb & jnp.int32(-1024)) | jidx
    cols = []
    base = b * N
    for _ in range(K):
        m = jnp.min(keys, axis=1, keepdims=True)
        cols.append((m & 1023) + base)
        keys = jnp.where(keys == m, jnp.int32(0x7FFFFFFF), keys)
    gidx_ref[...] = jnp.concatenate(cols, axis=1)


def _knn_call(xyzs, xyzt):
    return pl.pallas_call(
        _knn_body,
        grid=(B * NBLK_A,),
        in_specs=[
            pl.BlockSpec((1, ROWS_A, 3), lambda s: (s // NBLK_A, s % NBLK_A, 0)),
            pl.BlockSpec((1, 3, N), lambda s: (s // NBLK_A, 0, 0)),
        ],
        out_specs=pl.BlockSpec((ROWS_A, K), lambda s: (s, 0)),
        out_shape=jax.ShapeDtypeStruct((BN, K), jnp.int32),
    )(xyzs, xyzt)


def _pq_body(xt_ref, p_ref, q_ref, tp_s, tq_s):
    x = xt_ref[...]
    one = jnp.ones((3, BN), jnp.float32)
    tp_s[0:3, :] = x
    tp_s[3:6, :] = one
    tq_s[0:3, :] = one
    tq_s[3:6, :] = x
    for fi, f in enumerate(_FREQS):
        s = jnp.sin(x * f)
        c = jnp.cos(x * f)
        base = 6 + 12 * fi
        tp_s[base:base + 3, :] = s
        tp_s[base + 3:base + 6, :] = c
        tp_s[base + 6:base + 9, :] = c
        tp_s[base + 9:base + 12, :] = s
        tq_s[base:base + 3, :] = c
        tq_s[base + 3:base + 6, :] = s
        tq_s[base + 6:base + 9, :] = c
        tq_s[base + 9:base + 12, :] = s
    zpad = jnp.zeros((QW - 2 * NM, BN), jnp.float32)
    tp_s[2 * NM:, :] = zpad
    tq_s[2 * NM:, :] = zpad
    p_ref[...] = tp_s[...].T
    q_ref[...] = tq_s[...].T


def _pq_call(xt3):
    return pl.pallas_call(
        _pq_body,
        out_shape=[
            jax.ShapeDtypeStruct((BN, QW), jnp.float32),
            jax.ShapeDtypeStruct((BN, QW), jnp.float32),
        ],
        scratch_shapes=[
            pltpu.VMEM((QW, BN), jnp.float32),
            pltpu.VMEM((QW, BN), jnp.float32),
        ],
    )(xt3)


ROWS_B = 256
NSTEP_B = BN // ROWS_B


def _pack_pair(lo, hi):
    lo16 = lax.bitcast_convert_type(
        lo.astype(jnp.bfloat16).astype(jnp.float32), jnp.int32)
    hi16 = lax.bitcast_convert_type(
        hi.astype(jnp.bfloat16).astype(jnp.float32), jnp.int32)
    return (hi16 & jnp.int32(-65536)) | lax.shift_right_logical(lo16, 16)


def _unpack_pair(x):
    lo = lax.bitcast_convert_type(lax.shift_left(x, 16), jnp.float32)
    hi = lax.bitcast_convert_type(x & jnp.int32(-65536), jnp.float32)
    return lo, hi


def _proj_body(feats_ref, q66_ref, wq_ref, wk_ref, wv_ref, g1w_ref, g1b_ref,
               peb_ref, pewp_ref, smat_ref, qgc_ref, t1_ref, w66_ref,
               wqg_s, wkg_s, c1_s):
    step = pl.program_id(0)

    @pl.when(step == 0)
    def _init():
        g1w = g1w_ref[...]
        wqg_s[...] = wq_ref[...] @ g1w
        wkg_s[...] = wk_ref[...] @ g1w
        c1_s[...] = peb_ref[...] @ g1w + g1b_ref[...]
        smat = smat_ref[...]
        w66_ref[:, :D] = smat @ (pewp_ref[...] @ g1w)
        w66_ref[:, D:] = smat @ pewp_ref[...]

    f = feats_ref[...]
    qgc_ref[...] = f @ wqg_s[...] + c1_s[...]
    kg = f @ wkg_s[...]
    vf = f @ wv_ref[...]
    q66 = q66_ref[...]
    t1_ref[:, 0:128] = _pack_pair(kg[:, :128], kg[:, 128:])
    t1_ref[:, 128:256] = _pack_pair(vf[:, :128], vf[:, 128:])
    t1_ref[:, 256:320] = _pack_pair(q66[:, :64], q66[:, 64:])
    t1_ref[:, 320:384] = jnp.zeros((ROWS_B, 64), jnp.int32)


def _proj_call(feats2d, q66, wq, wk, wv, g1w, g1b, peb, pewp, smat):
    return pl.pallas_call(
        _proj_body,
        grid=(NSTEP_B,),
        in_specs=[
            pl.BlockSpec((ROWS_B, D), lambda s: (s, 0)),
            pl.BlockSpec((ROWS_B, QW), lambda s: (s, 0)),
            pl.BlockSpec((D, D), lambda s: (0, 0)),
            pl.BlockSpec((D, D), lambda s: (0, 0)),
            pl.BlockSpec((D, D), lambda s: (0, 0)),
            pl.BlockSpec((D, D), lambda s: (0, 0)),
            pl.BlockSpec((1, D), lambda s: (0, 0)),
            pl.BlockSpec((1, D), lambda s: (0, 0)),
            pl.BlockSpec((PEW, D), lambda s: (0, 0)),
            pl.BlockSpec((QW, PEW), lambda s: (0, 0)),
        ],
        out_specs=[
            pl.BlockSpec((ROWS_B, D), lambda s: (s, 0)),
            pl.BlockSpec((ROWS_B, TBW), lambda s: (s, 0)),
            pl.BlockSpec((QW, TW), lambda s: (0, 0)),
        ],
        out_shape=[
            jax.ShapeDtypeStruct((BN, D), jnp.float32),
            jax.ShapeDtypeStruct((BN, TBW), jnp.int32),
            jax.ShapeDtypeStruct((QW, TW), jnp.float32),
        ],
        scratch_shapes=[
            pltpu.VMEM((D, D), jnp.float32),
            pltpu.VMEM((D, D), jnp.float32),
            pltpu.VMEM((1, D), jnp.float32),
        ],
    )(feats2d, q66, wq, wk, wv, g1w, g1b, peb, pewp, smat)


NWORK = 32
CHUNK = 128
NCHUNK = E // (NWORK * CHUNK)


def _gather_body(t1_hbm, gidx_hbm, g1_out, idx_v, r1_v, sem1):
    c = lax.axis_index("c")
    s = lax.axis_index("s")
    wid = s * 2 + c

    def body(i, carry):
        base = (wid * NCHUNK + i) * CHUNK
        pltpu.sync_copy(gidx_hbm.at[pl.ds(base, CHUNK)], idx_v)
        pltpu.async_copy(t1_hbm.at[idx_v], r1_v, sem1).wait()
        pltpu.sync_copy(r1_v, g1_out.at[pl.ds(base, CHUNK)])
        return carry

    lax.fori_loop(0, NCHUNK, body, 0)


def _gather_call(t1, gidx):
    k = pl.kernel(
        _gather_body,
        out_type=jax.ShapeDtypeStruct((E, TBW), jnp.int32),
        mesh=plsc.VectorSubcoreMesh(core_axis_name="c", subcore_axis_name="s",
                                    num_cores=2, num_subcores=16),
        scratch_types=[
            pltpu.VMEM((CHUNK,), jnp.int32),
            pltpu.VMEM((CHUNK, TBW), jnp.int32),
            pltpu.SemaphoreType.DMA,
        ],
    )
    return k(t1, gidx)


P = 64
EP = P * K
NSTEP_C = BN // P


def _edge_body(qgc_ref, g1_ref, pfeat_ref, feats_ref, w66_ref,
               g2w_ref, peb_ref, res_ref, stats_ref, acc_s):
    step = pl.program_id(0)

    q = qgc_ref[...]
    qr = jnp.broadcast_to(q.reshape(P, 1, D), (P, K, D)).reshape(EP, D)
    pf = pfeat_ref[...]
    pfr = jnp.broadcast_to(pf.reshape(P, 1, QW), (P, K, QW)).reshape(EP, QW)
    g1i = g1_ref[...]
    kg_lo, kg_hi = _unpack_pair(g1i[:, 0:128])
    vf_lo, vf_hi = _unpack_pair(g1i[:, 128:256])
    qf_lo, qf_hi = _unpack_pair(g1i[:, 256:320])
    kg = jnp.concatenate([kg_lo, kg_hi], axis=1)
    vf = jnp.concatenate([vf_lo, vf_hi], axis=1)
    qfe = jnp.concatenate([qf_lo, qf_hi], axis=1)
    prod = pfr * qfe

    pe2 = jax.lax.dot(prod, w66_ref[...],
                      preferred_element_type=jnp.float32)

    h = jnp.maximum(qr - kg + pe2[:, :D], 0.0)
    a = jax.lax.dot(h, g2w_ref[...], preferred_element_type=jnp.float32)

    a3 = a.reshape(P, K, D)
    m = jnp.max(a3, axis=1, keepdims=True)
    e = jnp.exp(a3 - m)
    ssum = jnp.sum(e, axis=1)
    v3 = (vf + pe2[:, D:]).reshape(P, K, D)
    num = jnp.sum(e * v3, axis=1)
    res = num / ssum + peb_ref[...] + feats_ref[...]
    res_ref[...] = res

    @pl.when(step == 0)
    def _init():
        acc_s[...] = jnp.zeros((8, D), jnp.float32)

    acc_s[0:1, :] += jnp.sum(res, axis=0, keepdims=True)
    acc_s[1:2, :] += jnp.sum(res * res, axis=0, keepdims=True)

    @pl.when(step == NSTEP_C - 1)
    def _fin():
        stats_ref[...] = acc_s[...]


def _edge_call(qgc, g1, pfeat, feats2d, w66, g2w, peb):
    return pl.pallas_call(
        _edge_body,
        grid=(NSTEP_C,),
        in_specs=[
            pl.BlockSpec((P, D), lambda s: (s, 0)),
            pl.BlockSpec((EP, TBW), lambda s: (s, 0)),
            pl.BlockSpec((P, QW), lambda s: (s, 0)),
            pl.BlockSpec((P, D), lambda s: (s, 0)),
            pl.BlockSpec((QW, TW), lambda s: (0, 0)),
            pl.BlockSpec((D, D), lambda s: (0, 0)),
            pl.BlockSpec((1, D), lambda s: (0, 0)),
        ],
        out_specs=[
            pl.BlockSpec((P, D), lambda s: (s, 0)),
            pl.BlockSpec((8, D), lambda s: (0, 0)),
        ],
        out_shape=[
            jax.ShapeDtypeStruct((BN, D), jnp.float32),
            jax.ShapeDtypeStruct((8, D), jnp.float32),
        ],
        scratch_shapes=[pltpu.VMEM((8, D), jnp.float32)],
    )(qgc, g1, pfeat, feats2d, w66, g2w, peb)


ROWS_D = 256
NSTEP_D = BN // ROWS_D


def _bn_body(res_ref, stats_ref, bnw_ref, bnb_ref, out_ref):
    inv_n = jnp.float32(1.0 / BN)
    mean = stats_ref[0:1, :] * inv_n
    ex2 = stats_ref[1:2, :] * inv_n
    var = ex2 - mean * mean
    scale = lax.rsqrt(var + 1e-5) * bnw_ref[...]
    out_ref[...] = (res_ref[...] - mean) * scale + bnb_ref[...]


def _bn_call(res, stats, bnw, bnb):
    return pl.pallas_call(
        _bn_body,
        grid=(NSTEP_D,),
        in_specs=[
            pl.BlockSpec((ROWS_D, D), lambda s: (s, 0)),
            pl.BlockSpec((8, D), lambda s: (0, 0)),
            pl.BlockSpec((1, D), lambda s: (0, 0)),
            pl.BlockSpec((1, D), lambda s: (0, 0)),
        ],
        out_specs=pl.BlockSpec((ROWS_D, D), lambda s: (s, 0)),
        out_shape=jax.ShapeDtypeStruct((BN, D), jnp.float32),
    )(res, stats, bnw, bnb)


def kernel(xyz, feats, wq, wk, wv, g1w, g1b, g2w, g2b, pew, peb, bnw, bnb):
    del g2b
    xyzt = jnp.transpose(xyz, (0, 2, 1))
    gidx = _knn_call(xyz, xyzt).reshape(E)

    xt3 = xyzt.transpose(1, 0, 2).reshape(3, BN)
    pfeat, q66 = _pq_call(xt3)

    feats2d = feats.reshape(BN, D)
    pewp = jnp.zeros((PEW, D), jnp.float32).at[:NM].set(pew)
    smat = jnp.asarray(_S_NP)
    qgc, t1, w66 = _proj_call(feats2d, q66, wq, wk, wv, g1w,
                              g1b.reshape(1, D), peb.reshape(1, D),
                              pewp, smat)
    g1 = _gather_call(t1, gidx)

    res, stats = _edge_call(qgc, g1, pfeat, feats2d, w66, g2w,
                            peb.reshape(1, D))
    out = _bn_call(res, stats, bnw.reshape(1, D), bnb.reshape(1, D))
    return out.reshape(B, N, D)


# speedup vs baseline: 15.1101x; 1.0103x over previous
import jax
import jax.numpy as jnp
import numpy as np
from jax import lax
from jax.experimental import pallas as pl
from jax.experimental.pallas import tpu as pltpu
from jax.experimental.pallas import tpu_sc as plsc

D = 256
K = 16
NF = 5
B = 4
N = 1024
BN = B * N
E = BN * K
TW = 2 * D
QW = 128
TBW = 384
PEW = 40
NM = 33
_FREQS = [float(f) for f in np.linspace(1.0, 2.0 ** NF, NF)]

_S_NP = np.zeros((QW, PEW), np.float32)
for _c in range(3):
    _S_NP[_c, _c] = 1.0
    _S_NP[3 + _c, _c] = -1.0
for _fi in range(NF):
    _base = 6 + 12 * _fi
    for _c in range(3):
        _ms, _mc = 3 + 6 * _fi + _c, 3 + 6 * _fi + 3 + _c
        _S_NP[_base + _c, _ms] = 1.0
        _S_NP[_base + 3 + _c, _ms] = -1.0
        _S_NP[_base + 6 + _c, _mc] = 1.0
        _S_NP[_base + 9 + _c, _mc] = 1.0

ROWS_A = 1024
NBLK_A = N // ROWS_A


def _knn_body(xyzs_ref, xyzt_ref, gidx_ref):
    step = pl.program_id(0)
    b = step // NBLK_A
    xi = xyzs_ref[0]
    xjt = xyzt_ref[0]
    acc = jnp.zeros((ROWS_A, N), jnp.float32)
    for c in range(3):
        dif = xi[:, c:c + 1] - xjt[c:c + 1, :]
        acc = acc + dif * dif
    kb = lax.bitcast_convert_type(acc, jnp.int32)
    jidx = lax.broadcasted_iota(jnp.int32, (ROWS_A, N), 1)
    keys = (kb & jnp.int32(-1024)) | jidx
    cols = []
    base = b * N
    for _ in range(K):
        m = jnp.min(keys, axis=1, keepdims=True)
        cols.append((m & 1023) + base)
        keys = jnp.where(keys == m, jnp.int32(0x7FFFFFFF), keys)
    gidx_ref[...] = jnp.concatenate(cols, axis=1)


def _knn_call(xyzs, xyzt):
    return pl.pallas_call(
        _knn_body,
        grid=(B * NBLK_A,),
        in_specs=[
            pl.BlockSpec((1, ROWS_A, 3), lambda s: (s // NBLK_A, s % NBLK_A, 0)),
            pl.BlockSpec((1, 3, N), lambda s: (s // NBLK_A, 0, 0)),
        ],
        out_specs=pl.BlockSpec((ROWS_A, K), lambda s: (s, 0)),
        out_shape=jax.ShapeDtypeStruct((BN, K), jnp.int32),
    )(xyzs, xyzt)


def _pq_body(xt_ref, p_ref, q_ref, tp_s, tq_s):
    x = xt_ref[...]
    one = jnp.ones((3, BN), jnp.float32)
    tp_s[0:3, :] = x
    tp_s[3:6, :] = one
    tq_s[0:3, :] = one
    tq_s[3:6, :] = x
    for fi, f in enumerate(_FREQS):
        s = jnp.sin(x * f)
        c = jnp.cos(x * f)
        base = 6 + 12 * fi
        tp_s[base:base + 3, :] = s
        tp_s[base + 3:base + 6, :] = c
        tp_s[base + 6:base + 9, :] = c
        tp_s[base + 9:base + 12, :] = s
        tq_s[base:base + 3, :] = c
        tq_s[base + 3:base + 6, :] = s
        tq_s[base + 6:base + 9, :] = c
        tq_s[base + 9:base + 12, :] = s
    zpad = jnp.zeros((QW - 2 * NM, BN), jnp.float32)
    tp_s[2 * NM:, :] = zpad
    tq_s[2 * NM:, :] = zpad
    p_ref[...] = tp_s[...].T
    q_ref[...] = tq_s[...].T


def _pq_call(xt3):
    return pl.pallas_call(
        _pq_body,
        out_shape=[
            jax.ShapeDtypeStruct((BN, QW), jnp.float32),
            jax.ShapeDtypeStruct((BN, QW), jnp.float32),
        ],
        scratch_shapes=[
            pltpu.VMEM((QW, BN), jnp.float32),
            pltpu.VMEM((QW, BN), jnp.float32),
        ],
    )(xt3)


ROWS_B = 256
NSTEP_B = BN // ROWS_B


def _pack_pair(lo, hi):
    lo16 = lax.bitcast_convert_type(
        lo.astype(jnp.bfloat16).astype(jnp.float32), jnp.int32)
    hi16 = lax.bitcast_convert_type(
        hi.astype(jnp.bfloat16).astype(jnp.float32), jnp.int32)
    return (hi16 & jnp.int32(-65536)) | lax.shift_right_logical(lo16, 16)


def _unpack_pair(x):
    lo = lax.bitcast_convert_type(lax.shift_left(x, 16), jnp.float32)
    hi = lax.bitcast_convert_type(x & jnp.int32(-65536), jnp.float32)
    return lo, hi


def _proj_body(feats_ref, q66_ref, wq_ref, wk_ref, wv_ref, g1w_ref, g1b_ref,
               peb_ref, pewp_ref, smat_ref, qgc_ref, t1_ref, w66_ref,
               wqg_s, wkg_s, c1_s):
    step = pl.program_id(0)

    @pl.when(step == 0)
    def _init():
        g1w = g1w_ref[...]
        wqg_s[...] = wq_ref[...] @ g1w
        wkg_s[...] = wk_ref[...] @ g1w
        c1_s[...] = peb_ref[...] @ g1w + g1b_ref[...]
        smat = smat_ref[...]
        w66_ref[:, :D] = (smat @ (pewp_ref[...] @ g1w)).astype(jnp.bfloat16)
        w66_ref[:, D:] = (smat @ pewp_ref[...]).astype(jnp.bfloat16)

    f = feats_ref[...]
    qgc_ref[...] = f @ wqg_s[...] + c1_s[...]
    kg = f @ wkg_s[...]
    vf = f @ wv_ref[...]
    q66 = q66_ref[...]
    t1_ref[:, 0:128] = _pack_pair(kg[:, :128], kg[:, 128:])
    t1_ref[:, 128:256] = _pack_pair(vf[:, :128], vf[:, 128:])
    t1_ref[:, 256:320] = _pack_pair(q66[:, :64], q66[:, 64:])
    t1_ref[:, 320:384] = jnp.zeros((ROWS_B, 64), jnp.int32)


def _proj_call(feats2d, q66, wq, wk, wv, g1w, g1b, peb, pewp, smat):
    return pl.pallas_call(
        _proj_body,
        grid=(NSTEP_B,),
        in_specs=[
            pl.BlockSpec((ROWS_B, D), lambda s: (s, 0)),
            pl.BlockSpec((ROWS_B, QW), lambda s: (s, 0)),
            pl.BlockSpec((D, D), lambda s: (0, 0)),
            pl.BlockSpec((D, D), lambda s: (0, 0)),
            pl.BlockSpec((D, D), lambda s: (0, 0)),
            pl.BlockSpec((D, D), lambda s: (0, 0)),
            pl.BlockSpec((1, D), lambda s: (0, 0)),
            pl.BlockSpec((1, D), lambda s: (0, 0)),
            pl.BlockSpec((PEW, D), lambda s: (0, 0)),
            pl.BlockSpec((QW, PEW), lambda s: (0, 0)),
        ],
        out_specs=[
            pl.BlockSpec((ROWS_B, D), lambda s: (s, 0)),
            pl.BlockSpec((ROWS_B, TBW), lambda s: (s, 0)),
            pl.BlockSpec((QW, TW), lambda s: (0, 0)),
        ],
        out_shape=[
            jax.ShapeDtypeStruct((BN, D), jnp.float32),
            jax.ShapeDtypeStruct((BN, TBW), jnp.int32),
            jax.ShapeDtypeStruct((QW, TW), jnp.bfloat16),
        ],
        scratch_shapes=[
            pltpu.VMEM((D, D), jnp.float32),
            pltpu.VMEM((D, D), jnp.float32),
            pltpu.VMEM((1, D), jnp.float32),
        ],
    )(feats2d, q66, wq, wk, wv, g1w, g1b, peb, pewp, smat)


NWORK = 32
CHUNK = 128
NCHUNK = E // (NWORK * CHUNK)


def _gather_body(t1_hbm, gidx_hbm, g1_out, idx_v, r1_v, sem1):
    c = lax.axis_index("c")
    s = lax.axis_index("s")
    wid = s * 2 + c

    def body(i, carry):
        base = (wid * NCHUNK + i) * CHUNK
        pltpu.sync_copy(gidx_hbm.at[pl.ds(base, CHUNK)], idx_v)
        pltpu.async_copy(t1_hbm.at[idx_v], r1_v, sem1).wait()
        pltpu.sync_copy(r1_v, g1_out.at[pl.ds(base, CHUNK)])
        return carry

    lax.fori_loop(0, NCHUNK, body, 0)


def _gather_call(t1, gidx):
    k = pl.kernel(
        _gather_body,
        out_type=jax.ShapeDtypeStruct((E, TBW), jnp.int32),
        mesh=plsc.VectorSubcoreMesh(core_axis_name="c", subcore_axis_name="s",
                                    num_cores=2, num_subcores=16),
        scratch_types=[
            pltpu.VMEM((CHUNK,), jnp.int32),
            pltpu.VMEM((CHUNK, TBW), jnp.int32),
            pltpu.SemaphoreType.DMA,
        ],
    )
    return k(t1, gidx)


P = 64
EP = P * K
NSTEP_C = BN // P


def _edge_body(qgc_ref, g1_ref, pfeat_ref, feats_ref, w66_ref,
               g2w_ref, peb_ref, res_ref, stats_ref, acc_s):
    step = pl.program_id(0)

    q = qgc_ref[...]
    qr = jnp.broadcast_to(q.reshape(P, 1, D), (P, K, D)).reshape(EP, D)
    pf = pfeat_ref[...]
    pfr = jnp.broadcast_to(pf.reshape(P, 1, QW), (P, K, QW)).reshape(EP, QW)
    g1i = g1_ref[...]
    kg_lo, kg_hi = _unpack_pair(g1i[:, 0:128])
    vf_lo, vf_hi = _unpack_pair(g1i[:, 128:256])
    qf_lo, qf_hi = _unpack_pair(g1i[:, 256:320])
    kg = jnp.concatenate([kg_lo, kg_hi], axis=1)
    vf = jnp.concatenate([vf_lo, vf_hi], axis=1)
    qfe = jnp.concatenate([qf_lo, qf_hi], axis=1)
    prod = (pfr * qfe).astype(jnp.bfloat16)

    pe2 = jax.lax.dot(prod, w66_ref[...],
                      preferred_element_type=jnp.float32)

    h = jnp.maximum(qr - kg + pe2[:, :D], 0.0).astype(jnp.bfloat16)
    a = jax.lax.dot(h, g2w_ref[...], preferred_element_type=jnp.float32)

    a3 = a.reshape(P, K, D)
    m = jnp.max(a3, axis=1, keepdims=True)
    e = jnp.exp(a3 - m)
    ssum = jnp.sum(e, axis=1)
    v3 = (vf + pe2[:, D:]).reshape(P, K, D)
    num = jnp.sum(e * v3, axis=1)
    res = num / ssum + peb_ref[...] + feats_ref[...]
    res_ref[...] = res

    @pl.when(step == 0)
    def _init():
        acc_s[...] = jnp.zeros((8, D), jnp.float32)

    acc_s[0:1, :] += jnp.sum(res, axis=0, keepdims=True)
    acc_s[1:2, :] += jnp.sum(res * res, axis=0, keepdims=True)

    @pl.when(step == NSTEP_C - 1)
    def _fin():
        stats_ref[...] = acc_s[...]


def _edge_call(qgc, g1, pfeat, feats2d, w66, g2w, peb):
    return pl.pallas_call(
        _edge_body,
        grid=(NSTEP_C,),
        in_specs=[
            pl.BlockSpec((P, D), lambda s: (s, 0)),
            pl.BlockSpec((EP, TBW), lambda s: (s, 0)),
            pl.BlockSpec((P, QW), lambda s: (s, 0)),
            pl.BlockSpec((P, D), lambda s: (s, 0)),
            pl.BlockSpec((QW, TW), lambda s: (0, 0)),
            pl.BlockSpec((D, D), lambda s: (0, 0)),
            pl.BlockSpec((1, D), lambda s: (0, 0)),
        ],
        out_specs=[
            pl.BlockSpec((P, D), lambda s: (s, 0)),
            pl.BlockSpec((8, D), lambda s: (0, 0)),
        ],
        out_shape=[
            jax.ShapeDtypeStruct((BN, D), jnp.float32),
            jax.ShapeDtypeStruct((8, D), jnp.float32),
        ],
        scratch_shapes=[pltpu.VMEM((8, D), jnp.float32)],
    )(qgc, g1, pfeat, feats2d, w66, g2w, peb)


ROWS_D = 512
NSTEP_D = BN // ROWS_D


def _bn_body(res_ref, stats_ref, bnw_ref, bnb_ref, out_ref):
    inv_n = jnp.float32(1.0 / BN)
    mean = stats_ref[0:1, :] * inv_n
    ex2 = stats_ref[1:2, :] * inv_n
    var = ex2 - mean * mean
    scale = lax.rsqrt(var + 1e-5) * bnw_ref[...]
    out_ref[...] = (res_ref[...] - mean) * scale + bnb_ref[...]


def _bn_call(res, stats, bnw, bnb):
    return pl.pallas_call(
        _bn_body,
        grid=(NSTEP_D,),
        in_specs=[
            pl.BlockSpec((ROWS_D, D), lambda s: (s, 0)),
            pl.BlockSpec((8, D), lambda s: (0, 0)),
            pl.BlockSpec((1, D), lambda s: (0, 0)),
            pl.BlockSpec((1, D), lambda s: (0, 0)),
        ],
        out_specs=pl.BlockSpec((ROWS_D, D), lambda s: (s, 0)),
        out_shape=jax.ShapeDtypeStruct((BN, D), jnp.float32),
    )(res, stats, bnw, bnb)


def kernel(xyz, feats, wq, wk, wv, g1w, g1b, g2w, g2b, pew, peb, bnw, bnb):
    del g2b
    xyzt = jnp.transpose(xyz, (0, 2, 1))
    gidx = _knn_call(xyz, xyzt).reshape(E)

    xt3 = xyzt.transpose(1, 0, 2).reshape(3, BN)
    pfeat, q66 = _pq_call(xt3)

    feats2d = feats.reshape(BN, D)
    pewp = jnp.zeros((PEW, D), jnp.float32).at[:NM].set(pew)
    smat = jnp.asarray(_S_NP)
    qgc, t1, w66 = _proj_call(feats2d, q66, wq, wk, wv, g1w,
                              g1b.reshape(1, D), peb.reshape(1, D),
                              pewp, smat)
    g1 = _gather_call(t1, gidx)

    res, stats = _edge_call(qgc, g1, pfeat, feats2d, w66,
                            g2w.astype(jnp.bfloat16), peb.reshape(1, D))
    out = _bn_call(res, stats, bnw.reshape(1, D), bnb.reshape(1, D))
    return out.reshape(B, N, D)


# speedup vs baseline: 17.7947x; 1.1777x over previous
import jax
import jax.numpy as jnp
import numpy as np
from jax import lax
from jax.experimental import pallas as pl
from jax.experimental.pallas import tpu as pltpu
from jax.experimental.pallas import tpu_sc as plsc

D = 256
K = 16
NF = 5
B = 4
N = 1024
BN = B * N
E = BN * K
TW = 2 * D
QW = 128
TBW = 384
PEW = 40
NM = 33
_FREQS = [float(f) for f in np.linspace(1.0, 2.0 ** NF, NF)]

_S_NP = np.zeros((QW, PEW), np.float32)
for _c in range(3):
    _S_NP[_c, _c] = 1.0
    _S_NP[3 + _c, _c] = -1.0
for _fi in range(NF):
    _base = 6 + 12 * _fi
    for _c in range(3):
        _ms, _mc = 3 + 6 * _fi + _c, 3 + 6 * _fi + 3 + _c
        _S_NP[_base + _c, _ms] = 1.0
        _S_NP[_base + 3 + _c, _ms] = -1.0
        _S_NP[_base + 6 + _c, _mc] = 1.0
        _S_NP[_base + 9 + _c, _mc] = 1.0

ROWS_A = 1024
NBLK_A = N // ROWS_A


def _knn_body(bo, xyzs_ref, xyzt_ref, gidx_ref):
    step = pl.program_id(0)
    b = step // NBLK_A + bo
    xi = xyzs_ref[0]
    xjt = xyzt_ref[0]
    acc = jnp.zeros((ROWS_A, N), jnp.float32)
    for c in range(3):
        dif = xi[:, c:c + 1] - xjt[c:c + 1, :]
        acc = acc + dif * dif
    kb = lax.bitcast_convert_type(acc, jnp.int32)
    jidx = lax.broadcasted_iota(jnp.int32, (ROWS_A, N), 1)
    keys = (kb & jnp.int32(-1024)) | jidx
    cols = []
    base = b * N
    for _ in range(K):
        m = jnp.min(keys, axis=1, keepdims=True)
        cols.append((m & 1023) + base)
        keys = jnp.where(keys == m, jnp.int32(0x7FFFFFFF), keys)
    gidx_ref[...] = jnp.concatenate(cols, axis=1)


def _knn_call(xyzs, xyzt, bo, nb):
    import functools as _ft
    return pl.pallas_call(
        _ft.partial(_knn_body, bo),
        grid=(nb * NBLK_A,),
        in_specs=[
            pl.BlockSpec((1, ROWS_A, 3),
                         lambda s: (bo + s // NBLK_A, s % NBLK_A, 0)),
            pl.BlockSpec((1, 3, N), lambda s: (bo + s // NBLK_A, 0, 0)),
        ],
        out_specs=pl.BlockSpec((ROWS_A, K), lambda s: (s, 0)),
        out_shape=jax.ShapeDtypeStruct((nb * N // ROWS_A * ROWS_A, K),
                                       jnp.int32),
    )(xyzs, xyzt)


def _pq_body(xt_ref, p_ref, q_ref, tp_s, tq_s):
    x = xt_ref[...]
    one = jnp.ones((3, BN), jnp.float32)
    tp_s[0:3, :] = x
    tp_s[3:6, :] = one
    tq_s[0:3, :] = one
    tq_s[3:6, :] = x
    for fi, f in enumerate(_FREQS):
        s = jnp.sin(x * f)
        c = jnp.cos(x * f)
        base = 6 + 12 * fi
        tp_s[base:base + 3, :] = s
        tp_s[base + 3:base + 6, :] = c
        tp_s[base + 6:base + 9, :] = c
        tp_s[base + 9:base + 12, :] = s
        tq_s[base:base + 3, :] = c
        tq_s[base + 3:base + 6, :] = s
        tq_s[base + 6:base + 9, :] = c
        tq_s[base + 9:base + 12, :] = s
    zpad = jnp.zeros((QW - 2 * NM, BN), jnp.float32)
    tp_s[2 * NM:, :] = zpad
    tq_s[2 * NM:, :] = zpad
    p_ref[...] = tp_s[...].T
    q_ref[...] = tq_s[...].T


def _pq_call(xt3):
    return pl.pallas_call(
        _pq_body,
        out_shape=[
            jax.ShapeDtypeStruct((BN, QW), jnp.float32),
            jax.ShapeDtypeStruct((BN, QW), jnp.float32),
        ],
        scratch_shapes=[
            pltpu.VMEM((QW, BN), jnp.float32),
            pltpu.VMEM((QW, BN), jnp.float32),
        ],
    )(xt3)


ROWS_B = 256
NSTEP_B = BN // ROWS_B


def _pack_pair(lo, hi):
    lo16 = lax.bitcast_convert_type(
        lo.astype(jnp.bfloat16).astype(jnp.float32), jnp.int32)
    hi16 = lax.bitcast_convert_type(
        hi.astype(jnp.bfloat16).astype(jnp.float32), jnp.int32)
    return (hi16 & jnp.int32(-65536)) | lax.shift_right_logical(lo16, 16)


def _unpack_pair(x):
    lo = lax.bitcast_convert_type(lax.shift_left(x, 16), jnp.float32)
    hi = lax.bitcast_convert_type(x & jnp.int32(-65536), jnp.float32)
    return lo, hi


def _proj_body(feats_ref, q66_ref, wq_ref, wk_ref, wv_ref, g1w_ref, g1b_ref,
               peb_ref, pewp_ref, smat_ref, qgc_ref, t1_ref, w66_ref,
               wqg_s, wkg_s, c1_s):
    step = pl.program_id(0)

    @pl.when(step == 0)
    def _init():
        g1w = g1w_ref[...]
        wqg_s[...] = wq_ref[...] @ g1w
        wkg_s[...] = wk_ref[...] @ g1w
        c1_s[...] = peb_ref[...] @ g1w + g1b_ref[...]
        smat = smat_ref[...]
        w66_ref[:, :D] = (smat @ (pewp_ref[...] @ g1w)).astype(jnp.bfloat16)
        w66_ref[:, D:] = (smat @ pewp_ref[...]).astype(jnp.bfloat16)

    f = feats_ref[...]
    qgc_ref[...] = f @ wqg_s[...] + c1_s[...]
    kg = f @ wkg_s[...]
    vf = f @ wv_ref[...]
    q66 = q66_ref[...]
    t1_ref[:, 0:128] = _pack_pair(kg[:, :128], kg[:, 128:])
    t1_ref[:, 128:256] = _pack_pair(vf[:, :128], vf[:, 128:])
    t1_ref[:, 256:320] = _pack_pair(q66[:, :64], q66[:, 64:])
    t1_ref[:, 320:384] = jnp.zeros((ROWS_B, 64), jnp.int32)


def _proj_call(feats2d, q66, wq, wk, wv, g1w, g1b, peb, pewp, smat):
    return pl.pallas_call(
        _proj_body,
        grid=(NSTEP_B,),
        in_specs=[
            pl.BlockSpec((ROWS_B, D), lambda s: (s, 0)),
            pl.BlockSpec((ROWS_B, QW), lambda s: (s, 0)),
            pl.BlockSpec((D, D), lambda s: (0, 0)),
            pl.BlockSpec((D, D), lambda s: (0, 0)),
            pl.BlockSpec((D, D), lambda s: (0, 0)),
            pl.BlockSpec((D, D), lambda s: (0, 0)),
            pl.BlockSpec((1, D), lambda s: (0, 0)),
            pl.BlockSpec((1, D), lambda s: (0, 0)),
            pl.BlockSpec((PEW, D), lambda s: (0, 0)),
            pl.BlockSpec((QW, PEW), lambda s: (0, 0)),
        ],
        out_specs=[
            pl.BlockSpec((ROWS_B, D), lambda s: (s, 0)),
            pl.BlockSpec((ROWS_B, TBW), lambda s: (s, 0)),
            pl.BlockSpec((QW, TW), lambda s: (0, 0)),
        ],
        out_shape=[
            jax.ShapeDtypeStruct((BN, D), jnp.float32),
            jax.ShapeDtypeStruct((BN, TBW), jnp.int32),
            jax.ShapeDtypeStruct((QW, TW), jnp.bfloat16),
        ],
        scratch_shapes=[
            pltpu.VMEM((D, D), jnp.float32),
            pltpu.VMEM((D, D), jnp.float32),
            pltpu.VMEM((1, D), jnp.float32),
        ],
    )(feats2d, q66, wq, wk, wv, g1w, g1b, peb, pewp, smat)


NWORK = 32
CHUNK = 128
NCHUNK = E // (NWORK * CHUNK)


def _gather_body(nchunk, t1_hbm, gidx_hbm, g1_out, idx_v, r1_v, sem1):
    c = lax.axis_index("c")
    s = lax.axis_index("s")
    wid = s * 2 + c

    def body(i, carry):
        base = (wid * nchunk + i) * CHUNK
        pltpu.sync_copy(gidx_hbm.at[pl.ds(base, CHUNK)], idx_v)
        pltpu.async_copy(t1_hbm.at[idx_v], r1_v, sem1).wait()
        pltpu.sync_copy(r1_v, g1_out.at[pl.ds(base, CHUNK)])
        return carry

    lax.fori_loop(0, nchunk, body, 0)


def _gather_call(t1, gidx):
    import functools as _ft
    rows = gidx.shape[0]
    k = pl.kernel(
        _ft.partial(_gather_body, rows // (NWORK * CHUNK)),
        out_type=jax.ShapeDtypeStruct((rows, TBW), jnp.int32),
        mesh=plsc.VectorSubcoreMesh(core_axis_name="c", subcore_axis_name="s",
                                    num_cores=2, num_subcores=16),
        scratch_types=[
            pltpu.VMEM((CHUNK,), jnp.int32),
            pltpu.VMEM((CHUNK, TBW), jnp.int32),
            pltpu.SemaphoreType.DMA,
        ],
    )
    return k(t1, gidx)


P = 64
EP = P * K
NSTEP_C = BN // P


def _edge_body(nstep, qgc_ref, g1_ref, pfeat_ref, feats_ref, w66_ref,
               g2w_ref, peb_ref, res_ref, stats_ref, acc_s):
    step = pl.program_id(0)

    q = qgc_ref[...]
    qr = jnp.broadcast_to(q.reshape(P, 1, D), (P, K, D)).reshape(EP, D)
    pf = pfeat_ref[...]
    pfr = jnp.broadcast_to(pf.reshape(P, 1, QW), (P, K, QW)).reshape(EP, QW)
    g1i = g1_ref[...]
    kg_lo, kg_hi = _unpack_pair(g1i[:, 0:128])
    vf_lo, vf_hi = _unpack_pair(g1i[:, 128:256])
    qf_lo, qf_hi = _unpack_pair(g1i[:, 256:320])
    kg = jnp.concatenate([kg_lo, kg_hi], axis=1)
    vf = jnp.concatenate([vf_lo, vf_hi], axis=1)
    qfe = jnp.concatenate([qf_lo, qf_hi], axis=1)
    prod = (pfr * qfe).astype(jnp.bfloat16)

    pe2 = jax.lax.dot(prod, w66_ref[...],
                      preferred_element_type=jnp.float32)

    h = jnp.maximum(qr - kg + pe2[:, :D], 0.0).astype(jnp.bfloat16)
    a = jax.lax.dot(h, g2w_ref[...], preferred_element_type=jnp.float32)

    a3 = a.reshape(P, K, D)
    m = jnp.max(a3, axis=1, keepdims=True)
    e = jnp.exp(a3 - m)
    ssum = jnp.sum(e, axis=1)
    v3 = (vf + pe2[:, D:]).reshape(P, K, D)
    num = jnp.sum(e * v3, axis=1)
    res = num / ssum + peb_ref[...] + feats_ref[...]
    res_ref[...] = res

    @pl.when(step == 0)
    def _init():
        acc_s[...] = jnp.zeros((8, D), jnp.float32)

    acc_s[0:1, :] += jnp.sum(res, axis=0, keepdims=True)
    acc_s[1:2, :] += jnp.sum(res * res, axis=0, keepdims=True)

    @pl.when(step == nstep - 1)
    def _fin():
        stats_ref[...] = acc_s[...]


def _edge_call(qgc, g1, pfeat, feats2d, w66, g2w, peb, off, nstep):
    import functools as _ft
    return pl.pallas_call(
        _ft.partial(_edge_body, nstep),
        grid=(nstep,),
        in_specs=[
            pl.BlockSpec((P, D), lambda s: (off + s, 0)),
            pl.BlockSpec((EP, TBW), lambda s: (s, 0)),
            pl.BlockSpec((P, QW), lambda s: (off + s, 0)),
            pl.BlockSpec((P, D), lambda s: (off + s, 0)),
            pl.BlockSpec((QW, TW), lambda s: (0, 0)),
            pl.BlockSpec((D, D), lambda s: (0, 0)),
            pl.BlockSpec((1, D), lambda s: (0, 0)),
        ],
        out_specs=[
            pl.BlockSpec((P, D), lambda s: (s, 0)),
            pl.BlockSpec((8, D), lambda s: (0, 0)),
        ],
        out_shape=[
            jax.ShapeDtypeStruct((nstep * P, D), jnp.float32),
            jax.ShapeDtypeStruct((8, D), jnp.float32),
        ],
        scratch_shapes=[pltpu.VMEM((8, D), jnp.float32)],
    )(qgc, g1, pfeat, feats2d, w66, g2w, peb)


ROWS_D = 512
NSTEP_D = BN // ROWS_D


def _bn_body(res_ref, stats_a_ref, stats_b_ref, bnw_ref, bnb_ref, out_ref):
    inv_n = jnp.float32(1.0 / BN)
    stats = stats_a_ref[...] + stats_b_ref[...]
    mean = stats[0:1, :] * inv_n
    ex2 = stats[1:2, :] * inv_n
    var = ex2 - mean * mean
    scale = lax.rsqrt(var + 1e-5) * bnw_ref[...]
    out_ref[...] = (res_ref[...] - mean) * scale + bnb_ref[...]


def _bn_call(res, stats_a, stats_b, bnw, bnb):
    nstep = res.shape[0] // ROWS_D
    return pl.pallas_call(
        _bn_body,
        grid=(nstep,),
        in_specs=[
            pl.BlockSpec((ROWS_D, D), lambda s: (s, 0)),
            pl.BlockSpec((8, D), lambda s: (0, 0)),
            pl.BlockSpec((8, D), lambda s: (0, 0)),
            pl.BlockSpec((1, D), lambda s: (0, 0)),
            pl.BlockSpec((1, D), lambda s: (0, 0)),
        ],
        out_specs=pl.BlockSpec((ROWS_D, D), lambda s: (s, 0)),
        out_shape=jax.ShapeDtypeStruct((res.shape[0], D), jnp.float32),
    )(res, stats_a, stats_b, bnw, bnb)


def kernel(xyz, feats, wq, wk, wv, g1w, g1b, g2w, g2b, pew, peb, bnw, bnb):
    del g2b
    xyzt = jnp.transpose(xyz, (0, 2, 1))
    xt3 = xyzt.transpose(1, 0, 2).reshape(3, BN)
    pfeat, q66 = _pq_call(xt3)

    feats2d = feats.reshape(BN, D)
    pewp = jnp.zeros((PEW, D), jnp.float32).at[:NM].set(pew)
    smat = jnp.asarray(_S_NP)
    qgc, t1, w66 = _proj_call(feats2d, q66, wq, wk, wv, g1w,
                              g1b.reshape(1, D), peb.reshape(1, D),
                              pewp, smat)

    hb = B // 2
    g2wb = g2w.astype(jnp.bfloat16)
    pebr = peb.reshape(1, D)
    gidx0 = _knn_call(xyz, xyzt, 0, hb).reshape(E // 2)
    g1h0 = _gather_call(t1, gidx0)
    gidx1 = _knn_call(xyz, xyzt, hb, hb).reshape(E // 2)
    g1h1 = _gather_call(t1, gidx1)
    ns = NSTEP_C // 2
    res0, stats0 = _edge_call(qgc, g1h0, pfeat, feats2d, w66, g2wb, pebr,
                              0, ns)
    res1, stats1 = _edge_call(qgc, g1h1, pfeat, feats2d, w66, g2wb, pebr,
                              ns, ns)
    out0 = _bn_call(res0, stats0, stats1, bnw.reshape(1, D),
                    bnb.reshape(1, D))
    out1 = _bn_call(res1, stats0, stats1, bnw.reshape(1, D),
                    bnb.reshape(1, D))
    return jnp.concatenate([out0, out1], axis=0).reshape(B, N, D)
